# baseline (speedup 1.0000x reference)
import functools

import jax
import jax.numpy as jnp
from jax import lax
from jax.experimental import pallas as pl
from jax.experimental.pallas import tpu as pltpu
from jax.experimental.pallas import tpu_sc as plsc

B = 16384
EMB = 64
NC = 2
NS = 16
NW = NC * NS
BPW = B // NW
CHUNK = 128
NCH = BPW // CHUNK


def _sc_gather_pair(center_emb, context_emb, cid2d, xid2d):
    mesh = plsc.VectorSubcoreMesh(core_axis_name="c", subcore_axis_name="s")

    @functools.partial(
        pl.kernel,
        mesh=mesh,
        out_type=[
            jax.ShapeDtypeStruct((B, EMB), jnp.float32),
            jax.ShapeDtypeStruct((B, EMB), jnp.float32),
        ],
        scratch_types=[
            pltpu.VMEM((NCH, CHUNK), jnp.int32),
            pltpu.VMEM((NCH, CHUNK), jnp.int32),
            pltpu.VMEM((BPW, EMB), jnp.float32),
            pltpu.VMEM((BPW, EMB), jnp.float32),
            pltpu.SemaphoreType.DMA,
        ],
        compiler_params=pltpu.CompilerParams(use_tc_tiling_on_sc=False),
    )
    def k(cen_hbm, ctx_hbm, cid_hbm, xid_hbm, out_c_hbm, out_x_hbm,
          cid_v, xid_v, rows_c, rows_x, sem):
        wid = lax.axis_index("s") * NC + lax.axis_index("c")
        base = wid * BPW
        pltpu.sync_copy(cid_hbm.at[pl.ds(wid * NCH, NCH)], cid_v)
        pltpu.sync_copy(xid_hbm.at[pl.ds(wid * NCH, NCH)], xid_v)
        copies = []
        for j in range(NCH):
            dst = rows_c.at[pl.ds(j * CHUNK, CHUNK)]
            copies.append(pltpu.async_copy(cen_hbm.at[cid_v.at[j]], dst, sem))
        for j in range(NCH):
            dst = rows_x.at[pl.ds(j * CHUNK, CHUNK)]
            copies.append(pltpu.async_copy(ctx_hbm.at[xid_v.at[j]], dst, sem))
        for c in copies:
            c.wait()
        pltpu.sync_copy(rows_c, out_c_hbm.at[pl.ds(base, BPW)])
        pltpu.sync_copy(rows_x, out_x_hbm.at[pl.ds(base, BPW)])

    return k(center_emb, context_emb, cid2d, xid2d)


_TC_ROWS = 2048


def _tc_loss_body(c_ref, x_ref, y_ref, o_ref):
    s = jnp.sum(c_ref[...] * x_ref[...], axis=1)
    y = y_ref[...]
    t = jnp.maximum(s, 0.0) - s * y + jnp.log1p(jnp.exp(-jnp.abs(s)))
    part = jnp.sum(t).reshape(1, 1) * (1.0 / B)

    @pl.when(pl.program_id(0) == 0)
    def _():
        o_ref[...] = jnp.zeros((1, 1), jnp.float32)

    o_ref[...] += part


def _tc_loss(rows_c, rows_x, labels):
    grid = (B // _TC_ROWS,)
    return pl.pallas_call(
        _tc_loss_body,
        grid=grid,
        in_specs=[
            pl.BlockSpec((_TC_ROWS, EMB), lambda i: (i, 0)),
            pl.BlockSpec((_TC_ROWS, EMB), lambda i: (i, 0)),
            pl.BlockSpec((_TC_ROWS,), lambda i: (i,)),
        ],
        out_specs=pl.BlockSpec((1, 1), lambda i: (0, 0)),
        out_shape=jax.ShapeDtypeStruct((1, 1), jnp.float32),
    )(rows_c, rows_x, labels)


def kernel(batchContextId_int, batchCenterId_int, batchLabel_int, center_emb, context_emb):
    cid = batchCenterId_int.astype(jnp.int32).reshape(B // CHUNK, CHUNK)
    xid = batchContextId_int.astype(jnp.int32).reshape(B // CHUNK, CHUNK)
    rows_c, rows_x = _sc_gather_pair(center_emb, context_emb, cid, xid)
    loss = _tc_loss(rows_c, rows_x, batchLabel_int.astype(jnp.float32))
    return loss[0, 0]


# speedup vs baseline: 1.0011x; 1.0011x over previous
import functools

import jax
import jax.numpy as jnp
from jax import lax
from jax.experimental import pallas as pl
from jax.experimental.pallas import tpu as pltpu
from jax.experimental.pallas import tpu_sc as plsc

B = 16384
EMB = 64
W = 2 * EMB
NC = 2
NS = 16
NW = NC * NS
BPW = B // NW
CHUNK = 128
NCH = BPW // CHUNK


def _sc_gather_pair(cen2, ctx2, cidh, xidh):
    mesh = plsc.VectorSubcoreMesh(core_axis_name="c", subcore_axis_name="s")

    @functools.partial(
        pl.kernel,
        mesh=mesh,
        out_type=[
            jax.ShapeDtypeStruct((B, W), jnp.float32),
            jax.ShapeDtypeStruct((B, W), jnp.float32),
        ],
        scratch_types=[
            pltpu.VMEM((BPW,), jnp.int32),
            pltpu.VMEM((BPW,), jnp.int32),
            pltpu.VMEM((2, CHUNK, W), jnp.float32),
            pltpu.VMEM((2, CHUNK, W), jnp.float32),
            pltpu.SemaphoreType.DMA,
            pltpu.SemaphoreType.DMA,
        ],
    )
    def k(cen_hbm, ctx_hbm, cid_hbm, xid_hbm, out_c_hbm, out_x_hbm,
          cid_v, xid_v, buf_c, buf_x, sem0, sem1):
        wid = lax.axis_index("s") * NC + lax.axis_index("c")
        base = wid * BPW
        pltpu.sync_copy(cid_hbm.at[pl.ds(base, BPW)], cid_v)
        pltpu.sync_copy(xid_hbm.at[pl.ds(base, BPW)], xid_v)
        sems = (sem0, sem1)
        pend = [None, None]
        for j in range(NCH):
            b = j % 2
            if pend[b] is not None:
                for cp in pend[b]:
                    cp.wait()
                jo = j - 2
                pltpu.sync_copy(buf_c.at[b], out_c_hbm.at[pl.ds(base + jo * CHUNK, CHUNK)])
                pltpu.sync_copy(buf_x.at[b], out_x_hbm.at[pl.ds(base + jo * CHUNK, CHUNK)])
            idx_c = cid_v.at[pl.ds(j * CHUNK, CHUNK)]
            idx_x = xid_v.at[pl.ds(j * CHUNK, CHUNK)]
            pend[b] = (
                pltpu.async_copy(cen_hbm.at[idx_c], buf_c.at[b], sems[b]),
                pltpu.async_copy(ctx_hbm.at[idx_x], buf_x.at[b], sems[b]),
            )
        for j in (NCH - 2, NCH - 1):
            b = j % 2
            for cp in pend[b]:
                cp.wait()
            pltpu.sync_copy(buf_c.at[b], out_c_hbm.at[pl.ds(base + j * CHUNK, CHUNK)])
            pltpu.sync_copy(buf_x.at[b], out_x_hbm.at[pl.ds(base + j * CHUNK, CHUNK)])

    return k(cen2, ctx2, cidh, xidh)


_TC_ROWS = 2048


def _tc_loss_body(c_ref, x_ref, y_ref, pc_ref, px_ref, o_ref):
    c2 = c_ref[...]
    x2 = x_ref[...]
    csel = jnp.where(pc_ref[...][:, None] > 0.5, c2[:, EMB:], c2[:, :EMB])
    xsel = jnp.where(px_ref[...][:, None] > 0.5, x2[:, EMB:], x2[:, :EMB])
    s = jnp.sum(csel * xsel, axis=1)
    y = y_ref[...]
    t = jnp.maximum(s, 0.0) - s * y + jnp.log1p(jnp.exp(-jnp.abs(s)))
    part = jnp.sum(t).reshape(1, 1) * (1.0 / B)

    @pl.when(pl.program_id(0) == 0)
    def _():
        o_ref[...] = jnp.zeros((1, 1), jnp.float32)

    o_ref[...] += part


def _tc_loss(rows_c, rows_x, labels, pc, px):
    grid = (B // _TC_ROWS,)
    return pl.pallas_call(
        _tc_loss_body,
        grid=grid,
        in_specs=[
            pl.BlockSpec((_TC_ROWS, W), lambda i: (i, 0)),
            pl.BlockSpec((_TC_ROWS, W), lambda i: (i, 0)),
            pl.BlockSpec((_TC_ROWS,), lambda i: (i,)),
            pl.BlockSpec((_TC_ROWS,), lambda i: (i,)),
            pl.BlockSpec((_TC_ROWS,), lambda i: (i,)),
        ],
        out_specs=pl.BlockSpec((1, 1), lambda i: (0, 0)),
        out_shape=jax.ShapeDtypeStruct((1, 1), jnp.float32),
    )(rows_c, rows_x, labels, pc, px)


def kernel(batchContextId_int, batchCenterId_int, batchLabel_int, center_emb, context_emb):
    vocab = center_emb.shape[0]
    cid = batchCenterId_int.astype(jnp.int32)
    xid = batchContextId_int.astype(jnp.int32)
    cen2 = center_emb.reshape(vocab // 2, W)
    ctx2 = context_emb.reshape(vocab // 2, W)
    rows_c, rows_x = _sc_gather_pair(cen2, ctx2, cid >> 1, xid >> 1)
    pc = (cid & 1).astype(jnp.float32)
    px = (xid & 1).astype(jnp.float32)
    loss = _tc_loss(rows_c, rows_x, batchLabel_int.astype(jnp.float32), pc, px)
    return loss[0, 0]


# speedup vs baseline: 2.0984x; 2.0961x over previous
import functools

import jax
import jax.numpy as jnp
from jax import lax
from jax.experimental import pallas as pl
from jax.experimental.pallas import tpu as pltpu
from jax.experimental.pallas import tpu_sc as plsc

B = 16384
EMB = 64
NC = 2
NS = 16
NW = NC * NS
BPW = B // NW
L = 16
PROWS = B // 8


def _load_ids(id_hbm, id_v, base, sem):
    pltpu.async_copy(id_hbm.at[pl.ds(base, BPW)], id_v.at[pl.ds(0, BPW)], sem).wait()


def _gather_rows(tab_hbm, id_s, rows_v, sem, drain_hbm, drain_v):
    @pl.loop(0, BPW)
    def _(r):
        idx = id_s[pl.ds(r, L)][0]
        pltpu.make_async_copy(
            tab_hbm.at[pl.ds(idx, 1)], rows_v.at[pl.ds(r, 1)], sem
        ).start()

    pltpu.make_async_copy(
        drain_hbm.at[pl.ds(0, BPW * EMB)], drain_v, sem
    ).wait()


def _sc_gather_cen(cen, cid):
    mesh = plsc.VectorSubcoreMesh(core_axis_name="c", subcore_axis_name="s")

    @functools.partial(
        pl.kernel,
        mesh=mesh,
        out_type=jax.ShapeDtypeStruct((B * EMB,), jnp.float32),
        scratch_types=[
            pltpu.VMEM((BPW + L,), jnp.int32),
            pltpu.VMEM((BPW, EMB), jnp.float32),
            pltpu.VMEM((BPW * EMB,), jnp.float32),
            pltpu.SemaphoreType.DMA,
            pltpu.SemaphoreType.DMA,
        ],
    )
    def k(cen_hbm, cid_hbm, out_hbm, cid_v, rows_c, flat_v, sem_i, sem_c):
        wid = lax.axis_index("s") * NC + lax.axis_index("c")
        base = wid * BPW
        _load_ids(cid_hbm, cid_v, base, sem_i)
        _gather_rows(cen_hbm, cid_v, rows_c, sem_c, out_hbm, flat_v)

        @pl.loop(0, BPW)
        def _(r):
            for kk in range(EMB // L):
                flat_v[pl.ds(r * EMB + L * kk, L)] = rows_c[r, pl.ds(L * kk, L)]

        pltpu.sync_copy(flat_v, out_hbm.at[pl.ds(base * EMB, BPW * EMB)])

    return k(cen, cid)


def _sc_gather_ctx_dot(ctx, xid, cen_rows_flat):
    mesh = plsc.VectorSubcoreMesh(core_axis_name="c", subcore_axis_name="s")

    @functools.partial(
        pl.kernel,
        mesh=mesh,
        out_type=jax.ShapeDtypeStruct((B * L,), jnp.float32),
        scratch_types=[
            pltpu.VMEM((BPW + L,), jnp.int32),
            pltpu.VMEM((BPW, EMB), jnp.float32),
            pltpu.VMEM((BPW * EMB,), jnp.float32),
            pltpu.VMEM((BPW * L,), jnp.float32),
            pltpu.SemaphoreType.DMA,
            pltpu.SemaphoreType.DMA,
        ],
    )
    def k(ctx_hbm, xid_hbm, cen_flat_hbm, part_hbm,
          xid_v, rows_x, cen_flat_v, part_v, sem_i, sem_x):
        wid = lax.axis_index("s") * NC + lax.axis_index("c")
        base = wid * BPW
        _load_ids(xid_hbm, xid_v, base, sem_i)
        cf = pltpu.async_copy(
            cen_flat_hbm.at[pl.ds(base * EMB, BPW * EMB)], cen_flat_v, sem_i
        )
        cf.wait()
        _gather_rows(ctx_hbm, xid_v, rows_x, sem_x, cen_flat_hbm, cen_flat_v)

        @pl.loop(0, BPW)
        def _(r):
            acc = cen_flat_v[pl.ds(r * EMB, L)] * rows_x[r, pl.ds(0, L)]
            for kk in range(1, EMB // L):
                acc += (cen_flat_v[pl.ds(r * EMB + L * kk, L)]
                        * rows_x[r, pl.ds(L * kk, L)])
            part_v[pl.ds(r * L, L)] = acc

        pltpu.sync_copy(part_v, part_hbm.at[pl.ds(base * L, BPW * L)])

    return k(ctx, xid, cen_rows_flat)


def _tc_loss_body(p_ref, y_ref, o_ref):
    p = p_ref[...]
    s = jnp.sum(p.reshape(PROWS, 8, L), axis=2)
    y = y_ref[...]
    t = jnp.maximum(s, 0.0) - s * y + jnp.log1p(jnp.exp(-jnp.abs(s)))
    o_ref[...] = jnp.sum(t).reshape(1, 1) * (1.0 / B)


def _tc_loss(partials, labels2d):
    return pl.pallas_call(
        _tc_loss_body,
        out_shape=jax.ShapeDtypeStruct((1, 1), jnp.float32),
    )(partials, labels2d)


def kernel(batchContextId_int, batchCenterId_int, batchLabel_int, center_emb, context_emb):
    cid = batchCenterId_int.astype(jnp.int32)
    xid = batchContextId_int.astype(jnp.int32)
    cen_rows_flat = _sc_gather_cen(center_emb, cid)
    partials = _sc_gather_ctx_dot(context_emb, xid, cen_rows_flat).reshape(PROWS, 128)
    labels2d = batchLabel_int.astype(jnp.float32).reshape(PROWS, 8)
    loss = _tc_loss(partials, labels2d)
    return loss[0, 0]
